# baseline (speedup 1.0000x reference)
import jax
import jax.numpy as jnp
from jax import lax
from jax.experimental import pallas as pl
from jax.experimental.pallas import tpu as pltpu
from jax.experimental.pallas import tpu_sc as plsc

_NC = 2
_NS = 16
_NW = _NC * _NS
_C = 32
_LANES = 16


def _make_sc_add(R, S, D):
    rows_per_w = R // _NW
    n_chunks = rows_per_w // _C
    E = _C * D
    assert R % _NW == 0 and rows_per_w % _C == 0
    assert S % rows_per_w == 0
    mesh = plsc.VectorSubcoreMesh(
        core_axis_name="c", subcore_axis_name="s",
        num_cores=_NC, num_subcores=_NS,
    )

    def body(x_hbm, pos_hbm, out_hbm, X0, X1, P0, P1, sem_x, sem_p, sem_o):
        wid = lax.axis_index("s") * _NC + lax.axis_index("c")
        rbase = wid * rows_per_w
        sbase = lax.rem(rbase, S)
        Xs, Ps = [X0, X1], [P0, P1]

        def start_x(i):
            return pltpu.async_copy(
                x_hbm.at[pl.ds((rbase + i * _C) * D, E)], Xs[i % 2], sem_x)

        def start_p(i):
            return pltpu.async_copy(
                pos_hbm.at[pl.ds((sbase + i * _C) * D, E)], Ps[i % 2], sem_p)

        cp_x = {0: start_x(0)}
        cp_p = {0: start_p(0)}
        cp_o = {}
        for i in range(n_chunks):
            cur = i % 2
            if i + 1 < n_chunks:
                if i >= 1:
                    cp_o[i - 1].wait()
                cp_x[i + 1] = start_x(i + 1)
                cp_p[i + 1] = start_p(i + 1)
            cp_x[i].wait()
            cp_p[i].wait()
            Xc, Pc = Xs[cur], Ps[cur]

            @plsc.parallel_loop(0, E, _LANES, unroll=8)
            def addbody(j):
                Xc[pl.ds(j, _LANES)] = Xc[pl.ds(j, _LANES)] + Pc[pl.ds(j, _LANES)]

            cp_o[i] = pltpu.async_copy(
                Xs[cur], out_hbm.at[pl.ds((rbase + i * _C) * D, E)], sem_o)
        cp_o[n_chunks - 1].wait()

    return pl.kernel(
        body,
        out_type=jax.ShapeDtypeStruct((R * D,), jnp.float32),
        mesh=mesh,
        scratch_types=[
            pltpu.VMEM((E,), jnp.float32),
            pltpu.VMEM((E,), jnp.float32),
            pltpu.VMEM((E,), jnp.float32),
            pltpu.VMEM((E,), jnp.float32),
            pltpu.SemaphoreType.DMA,
            pltpu.SemaphoreType.DMA,
            pltpu.SemaphoreType.DMA,
        ],
    )


def kernel(x, pos_table):
    B, S, D = x.shape
    R = B * S
    out = _make_sc_add(R, S, D)(x.reshape(R * D), pos_table.reshape(S * D))
    return out.reshape(B, S, D)


# speedup vs baseline: 1.0904x; 1.0904x over previous
import jax
import jax.numpy as jnp
from jax import lax
from jax.experimental import pallas as pl
from jax.experimental.pallas import tpu as pltpu
from jax.experimental.pallas import tpu_sc as plsc

_NC = 2
_NS = 16
_NW = _NC * _NS
_C = 32
_LANES = 16


def _make_sc_add(B, S, D):
    s_per_w = S // _NW
    n_chunks = s_per_w // _C
    E = _C * D
    total = n_chunks * B
    assert S % _NW == 0 and s_per_w % _C == 0
    mesh = plsc.VectorSubcoreMesh(
        core_axis_name="c", subcore_axis_name="s",
        num_cores=_NC, num_subcores=_NS,
    )

    def body(x_hbm, pos_hbm, out_hbm, X0, X1, X2, P0, P1, sem_x, sem_p, sem_o):
        wid = lax.axis_index("s") * _NC + lax.axis_index("c")
        sbase = wid * s_per_w
        Xs, Ps = [X0, X1, X2], [P0, P1]

        def start_p(i):
            return pltpu.async_copy(
                pos_hbm.at[pl.ds((sbase + i * _C) * D, E)], Ps[i % 2], sem_p)

        def row0(step):
            i, b = divmod(step, B)
            return b * S + sbase + i * _C

        def start_x(step):
            return pltpu.async_copy(
                x_hbm.at[pl.ds(row0(step) * D, E)], Xs[step % 3], sem_x)

        cp_p = {0: start_p(0)}
        cp_x = {0: start_x(0), 1: start_x(1)}
        cp_o = {}
        step = 0
        for i in range(n_chunks):
            if i + 1 < n_chunks:
                cp_p[i + 1] = start_p(i + 1)
            for b in range(B):
                if step >= 2:
                    cp_o[step - 2].wait()
                if step + 2 < total:
                    cp_x[step + 2] = start_x(step + 2)
                if b == 0:
                    cp_p[i].wait()
                cp_x[step].wait()
                Xc, Pc = Xs[step % 3], Ps[i % 2]

                @plsc.parallel_loop(0, E, _LANES, unroll=8)
                def addbody(j):
                    Xc[pl.ds(j, _LANES)] = (
                        Xc[pl.ds(j, _LANES)] + Pc[pl.ds(j, _LANES)])

                cp_o[step] = pltpu.async_copy(
                    Xc, out_hbm.at[pl.ds(row0(step) * D, E)], sem_o)
                step += 1
        cp_o[total - 2].wait()
        cp_o[total - 1].wait()

    return pl.kernel(
        body,
        out_type=jax.ShapeDtypeStruct((B * S * D,), jnp.float32),
        mesh=mesh,
        scratch_types=[
            pltpu.VMEM((E,), jnp.float32),
            pltpu.VMEM((E,), jnp.float32),
            pltpu.VMEM((E,), jnp.float32),
            pltpu.VMEM((E,), jnp.float32),
            pltpu.VMEM((E,), jnp.float32),
            pltpu.SemaphoreType.DMA,
            pltpu.SemaphoreType.DMA,
            pltpu.SemaphoreType.DMA,
        ],
    )


def kernel(x, pos_table):
    B, S, D = x.shape
    out = _make_sc_add(B, S, D)(x.reshape(B * S * D), pos_table.reshape(S * D))
    return out.reshape(B, S, D)


# speedup vs baseline: 3.4100x; 3.1274x over previous
import jax
import jax.numpy as jnp
from jax import lax
from jax.experimental import pallas as pl
from jax.experimental.pallas import tpu as pltpu
from jax.experimental.pallas import tpu_sc as plsc

_NC = 2
_NS = 16
_NW = _NC * _NS
_C = 32
_LANES = 16


def _make_sc_add(B, S, D):
    s_per_w = S // _NW
    n_chunks = s_per_w // _C
    total = n_chunks * B
    assert S % _NW == 0 and s_per_w % _C == 0 and D % _LANES == 0
    mesh = plsc.VectorSubcoreMesh(
        core_axis_name="c", subcore_axis_name="s",
        num_cores=_NC, num_subcores=_NS,
    )

    def body(x_hbm, pos_hbm, out_hbm, X0, X1, X2, P0, P1, sem_x, sem_p, sem_o):
        wid = lax.axis_index("s") * _NC + lax.axis_index("c")
        sbase = wid * s_per_w
        Xs, Ps = [X0, X1, X2], [P0, P1]

        def start_p(i):
            return pltpu.async_copy(
                pos_hbm.at[pl.ds(sbase + i * _C, _C)], Ps[i % 2], sem_p)

        def row0(step):
            i, b = divmod(step, B)
            return b * S + sbase + i * _C

        def start_x(step):
            return pltpu.async_copy(
                x_hbm.at[pl.ds(row0(step), _C)], Xs[step % 3], sem_x)

        cp_p = {0: start_p(0)}
        cp_x = {0: start_x(0), 1: start_x(1)}
        cp_o = {}
        step = 0
        for i in range(n_chunks):
            if i + 1 < n_chunks:
                cp_p[i + 1] = start_p(i + 1)
            for b in range(B):
                if step >= 2:
                    cp_o[step - 2].wait()
                if step + 2 < total:
                    cp_x[step + 2] = start_x(step + 2)
                if b == 0:
                    cp_p[i].wait()
                cp_x[step].wait()
                Xc, Pc = Xs[step % 3], Ps[i % 2]

                @plsc.parallel_loop(0, _C * D, _LANES, unroll=8)
                def addbody(j):
                    r = j // D
                    c = j - r * D
                    Xc[r, pl.ds(c, _LANES)] = (
                        Xc[r, pl.ds(c, _LANES)] + Pc[r, pl.ds(c, _LANES)])

                cp_o[step] = pltpu.async_copy(
                    Xc, out_hbm.at[pl.ds(row0(step), _C)], sem_o)
                step += 1
        cp_o[total - 2].wait()
        cp_o[total - 1].wait()

    return pl.kernel(
        body,
        out_type=jax.ShapeDtypeStruct((B * S, D), jnp.float32),
        mesh=mesh,
        scratch_types=[
            pltpu.VMEM((_C, D), jnp.float32),
            pltpu.VMEM((_C, D), jnp.float32),
            pltpu.VMEM((_C, D), jnp.float32),
            pltpu.VMEM((_C, D), jnp.float32),
            pltpu.VMEM((_C, D), jnp.float32),
            pltpu.SemaphoreType.DMA,
            pltpu.SemaphoreType.DMA,
            pltpu.SemaphoreType.DMA,
        ],
    )


def kernel(x, pos_table):
    B, S, D = x.shape
    out = _make_sc_add(B, S, D)(x.reshape(B * S, D), pos_table)
    return out.reshape(B, S, D)


# speedup vs baseline: 3.4342x; 1.0071x over previous
import jax
import jax.numpy as jnp
from jax import lax
from jax.experimental import pallas as pl
from jax.experimental.pallas import tpu as pltpu
from jax.experimental.pallas import tpu_sc as plsc

_NC = 2
_NS = 16
_NW = _NC * _NS
_C = 32
_LANES = 16


def _make_sc_add(B, S, D):
    s_per_w = S // _NW
    n_chunks = s_per_w // _C
    total = n_chunks * B
    assert S % _NW == 0 and s_per_w % _C == 0 and D % _LANES == 0
    mesh = plsc.VectorSubcoreMesh(
        core_axis_name="c", subcore_axis_name="s",
        num_cores=_NC, num_subcores=_NS,
    )

    def body(x_hbm, pos_hbm, out_hbm, X0, X1, X2, P0, P1, sem_x, sem_p, sem_o):
        wid = lax.axis_index("s") * _NC + lax.axis_index("c")
        sbase = wid * s_per_w
        Xs, Ps = [X0, X1, X2], [P0, P1]

        def start_p(i):
            return pltpu.async_copy(
                pos_hbm.at[pl.ds(sbase + i * _C, _C)], Ps[i % 2], sem_p)

        def row0(step):
            i, b = divmod(step, B)
            return b * S + sbase + i * _C

        def start_x(step):
            return pltpu.async_copy(
                x_hbm.at[pl.ds(row0(step), _C)], Xs[step % 3], sem_x)

        cp_p = {0: start_p(0)}
        cp_x = {0: start_x(0), 1: start_x(1)}
        cp_o = {}
        step = 0
        for i in range(n_chunks):
            if i + 1 < n_chunks:
                cp_p[i + 1] = start_p(i + 1)
            for b in range(B):
                if step >= 2:
                    cp_o[step - 2].wait()
                if step + 2 < total:
                    cp_x[step + 2] = start_x(step + 2)
                if b == 0:
                    cp_p[i].wait()
                cp_x[step].wait()
                Xc, Pc = Xs[step % 3], Ps[i % 2]

                @plsc.parallel_loop(0, _C * D, _LANES, unroll=8)
                def addbody(j):
                    r = j // D
                    c = j - r * D
                    plsc.addupdate(
                        Xc.at[r, pl.ds(c, _LANES)], Pc[r, pl.ds(c, _LANES)])

                cp_o[step] = pltpu.async_copy(
                    Xc, out_hbm.at[pl.ds(row0(step), _C)], sem_o)
                step += 1
        cp_o[total - 2].wait()
        cp_o[total - 1].wait()

    return pl.kernel(
        body,
        out_type=jax.ShapeDtypeStruct((B * S, D), jnp.float32),
        mesh=mesh,
        scratch_types=[
            pltpu.VMEM((_C, D), jnp.float32),
            pltpu.VMEM((_C, D), jnp.float32),
            pltpu.VMEM((_C, D), jnp.float32),
            pltpu.VMEM((_C, D), jnp.float32),
            pltpu.VMEM((_C, D), jnp.float32),
            pltpu.SemaphoreType.DMA,
            pltpu.SemaphoreType.DMA,
            pltpu.SemaphoreType.DMA,
        ],
    )


def kernel(x, pos_table):
    B, S, D = x.shape
    out = _make_sc_add(B, S, D)(x.reshape(B * S, D), pos_table)
    return out.reshape(B, S, D)
